# baseline (speedup 1.0000x reference)
import jax
import jax.numpy as jnp
from jax.experimental import pallas as pl
from jax.experimental.pallas import tpu as pltpu

_N = 8192
_D = 768
_E = 8
_BT = 256


def _moe_body(x_ref, wg_ref, bg_ref, wt_ref, be_ref, out_ref):
    xb = x_ref[...]
    logits = jnp.dot(xb, wg_ref[...], preferred_element_type=jnp.float32)
    logits = logits + bg_ref[...]
    m = jnp.max(logits, axis=-1, keepdims=True)
    ex = jnp.exp(logits - m)
    p = ex / jnp.sum(ex, axis=-1, keepdims=True)
    i1 = jnp.argmax(logits, axis=-1)
    eids = jax.lax.broadcasted_iota(jnp.int32, logits.shape, 1)
    sel1 = eids == i1[:, None]
    masked = jnp.where(sel1, -jnp.inf, logits)
    i2 = jnp.argmax(masked, axis=-1)
    sel = sel1 | (eids == i2[:, None])
    w = jnp.where(sel, p, 0.0)
    xw = jnp.concatenate([w[:, e, None] * xb for e in range(_E)], axis=1)
    out = jnp.dot(xw, wt_ref[...], preferred_element_type=jnp.float32)
    out_ref[...] = out + jnp.dot(w, be_ref[...], preferred_element_type=jnp.float32)


@jax.jit
def _moe(x, W_tall, b_experts, W_gate, b_gate2d):
    grid = (_N // _BT,)
    return pl.pallas_call(
        _moe_body,
        grid=grid,
        in_specs=[
            pl.BlockSpec((_BT, _D), lambda i: (i, 0)),
            pl.BlockSpec((_D, _E), lambda i: (0, 0)),
            pl.BlockSpec((1, _E), lambda i: (0, 0)),
            pl.BlockSpec((_E * _D, _D), lambda i: (0, 0)),
            pl.BlockSpec((_E, _D), lambda i: (0, 0)),
        ],
        out_specs=pl.BlockSpec((_BT, _D), lambda i: (i, 0)),
        out_shape=jax.ShapeDtypeStruct((_N, _D), jnp.float32),
    )(x, W_gate, b_gate2d, W_tall, b_experts)


def kernel(x, W_experts, b_experts, W_gate, b_gate):
    return _moe(x, W_experts.reshape(_E * _D, _D), b_experts, W_gate,
                b_gate.reshape(1, _E))


# speedup vs baseline: 1.1334x; 1.1334x over previous
import jax
import jax.numpy as jnp
from jax.experimental import pallas as pl
from jax.experimental.pallas import tpu as pltpu

_N = 8192
_D = 768
_E = 8
_BT = 512


def _moe_body(x_ref, wg_ref, bg_ref, we_ref, be_ref, out_ref):
    xb = x_ref[...]
    logits = jnp.dot(xb, wg_ref[...], preferred_element_type=jnp.float32)
    logits = logits + bg_ref[...]
    m = jnp.max(logits, axis=-1, keepdims=True)
    ex = jnp.exp(logits - m)
    p = ex / jnp.sum(ex, axis=-1, keepdims=True)
    i1 = jnp.argmax(logits, axis=-1)
    eids = jax.lax.broadcasted_iota(jnp.int32, logits.shape, 1)
    sel1 = eids == i1[:, None]
    masked = jnp.where(sel1, -jnp.inf, logits)
    i2 = jnp.argmax(masked, axis=-1)
    sel = sel1 | (eids == i2[:, None])
    w = jnp.where(sel, p, 0.0)
    acc = jnp.dot(w, be_ref[...], preferred_element_type=jnp.float32)
    for e in range(_E):
        ye = jnp.dot(xb, we_ref[e], preferred_element_type=jnp.float32)
        acc = acc + w[:, e, None] * ye
    out_ref[...] = acc


@jax.jit
def _moe(x, W_experts, b_experts, W_gate, b_gate2d):
    grid = (_N // _BT,)
    return pl.pallas_call(
        _moe_body,
        grid=grid,
        in_specs=[
            pl.BlockSpec((_BT, _D), lambda i: (i, 0)),
            pl.BlockSpec((_D, _E), lambda i: (0, 0)),
            pl.BlockSpec((1, _E), lambda i: (0, 0)),
            pl.BlockSpec((_E, _D, _D), lambda i: (0, 0, 0)),
            pl.BlockSpec((_E, _D), lambda i: (0, 0)),
        ],
        out_specs=pl.BlockSpec((_BT, _D), lambda i: (i, 0)),
        out_shape=jax.ShapeDtypeStruct((_N, _D), jnp.float32),
    )(x, W_gate, b_gate2d, W_experts, b_experts)


def kernel(x, W_experts, b_experts, W_gate, b_gate):
    return _moe(x, W_experts, b_experts, W_gate, b_gate.reshape(1, _E))


# speedup vs baseline: 1.1767x; 1.0382x over previous
import jax
import jax.numpy as jnp
from jax.experimental import pallas as pl
from jax.experimental.pallas import tpu as pltpu

_N = 8192
_D = 768
_E = 8
_BT = 1024


def _moe_body(x_ref, wg_ref, bg_ref, we_ref, be_ref, out_ref):
    xb = x_ref[...]
    logits = jnp.dot(xb, wg_ref[...], preferred_element_type=jnp.float32)
    logits = logits + bg_ref[...]
    m = jnp.max(logits, axis=-1, keepdims=True)
    ex = jnp.exp(logits - m)
    p = ex / jnp.sum(ex, axis=-1, keepdims=True)
    i1 = jnp.argmax(logits, axis=-1)
    eids = jax.lax.broadcasted_iota(jnp.int32, logits.shape, 1)
    sel1 = eids == i1[:, None]
    masked = jnp.where(sel1, -jnp.inf, logits)
    i2 = jnp.argmax(masked, axis=-1)
    sel = sel1 | (eids == i2[:, None])
    w = jnp.where(sel, p, 0.0)
    acc = jnp.dot(w, be_ref[...], preferred_element_type=jnp.float32)
    for e in range(_E):
        ye = jnp.dot(xb, we_ref[e], preferred_element_type=jnp.float32)
        acc = acc + w[:, e, None] * ye
    out_ref[...] = acc


@jax.jit
def _moe(x, W_experts, b_experts, W_gate, b_gate2d):
    grid = (_N // _BT,)
    return pl.pallas_call(
        _moe_body,
        grid=grid,
        in_specs=[
            pl.BlockSpec((_BT, _D), lambda i: (i, 0)),
            pl.BlockSpec((_D, _E), lambda i: (0, 0)),
            pl.BlockSpec((1, _E), lambda i: (0, 0)),
            pl.BlockSpec((_E, _D, _D), lambda i: (0, 0, 0)),
            pl.BlockSpec((_E, _D), lambda i: (0, 0)),
        ],
        out_specs=pl.BlockSpec((_BT, _D), lambda i: (i, 0)),
        out_shape=jax.ShapeDtypeStruct((_N, _D), jnp.float32),
    )(x, W_gate, b_gate2d, W_experts, b_experts)


def kernel(x, W_experts, b_experts, W_gate, b_gate):
    return _moe(x, W_experts, b_experts, W_gate, b_gate.reshape(1, _E))
